# baseline (speedup 1.0000x reference)
import functools

import numpy as np
import jax
import jax.numpy as jnp
from jax import lax
from jax.experimental import pallas as pl
from jax.experimental.pallas import tpu as pltpu
from jax.experimental.pallas import tpu_sc as plsc

MAX_LEN = 200
MODEL_DIM = 64
NW = 32
NB = 4
LANES = 16
BT = 128
OVP = BT + 1


def _pe_np():
    pos = np.arange(MAX_LEN)[:, None]
    pe = pos / np.power(10000, 2.0 * np.arange(MODEL_DIM)[None, :] / MODEL_DIM)
    pe[:, 0::2] = np.sin(pe[:, 0::2])
    pe[:, 1::2] = np.cos(pe[:, 1::2])
    return pe.astype(np.float32)


def _sc_body(ipw, nbt, xn4_hbm, pe_hbm, table_hbm, out_hbm, idx_all, pe_v,
             i0, i1, i2, i3, r0, r1, r2, r3, o0, o1, o2, o3,
             is0, is1, is2, is3, gs0, gs1, gs2, gs3, os0, os1, os2, os3):
    del idx_all
    idxb = (i0, i1, i2, i3)
    rows = (r0, r1, r2, r3)
    outs = (o0, o1, o2, o3)
    isem = (is0, is1, is2, is3)
    gsem = (gs0, gs1, gs2, gs3)
    osem = (os0, os1, os2, os3)
    wid = lax.axis_index("s") * 2 + lax.axis_index("c")
    base = wid * ipw
    pltpu.sync_copy(pe_hbm, pe_v)
    iota = lax.iota(jnp.int32, LANES)
    dvec = [iota + (c * LANES) for c in range(MODEL_DIM // LANES)]

    def idx_src(i):
        g = base + i
        s, bt = g // nbt, g % nbt
        return xn4_hbm.at[s // 8, bt, s % 8]

    def fire_idx(i, sl):
        pltpu.async_copy(idx_src(i), idxb[sl], isem[sl])

    def wait_idx(i, sl):
        pltpu.make_async_copy(idx_src(i), idxb[sl], isem[sl]).wait()

    def fire_gather(i, sl):
        pltpu.async_copy(table_hbm.at[idxb[sl]], rows[sl], gsem[sl])

    def wait_gather(i, sl):
        pltpu.make_async_copy(table_hbm.at[idxb[sl]], rows[sl],
                              gsem[sl]).wait()

    def fire_out(i, sl):
        g = base + i
        s, bt = g // nbt, g % nbt
        for dt in range(MODEL_DIM // 8):
            pltpu.async_copy(outs[sl].at[pl.ds(dt * 8, 8), pl.ds(0, BT)],
                             out_hbm.at[s, dt, bt], osem[sl])

    def wait_out(i, sl):
        g = base + i
        s, bt = g // nbt, g % nbt
        for dt in range(MODEL_DIM // 8):
            pltpu.make_async_copy(outs[sl].at[pl.ds(dt * 8, 8), pl.ds(0, BT)],
                                  out_hbm.at[s, dt, bt], osem[sl]).wait()

    for sl in range(NB):
        fire_idx(sl, sl)
    def double_idx(sl):
        for c in range(BT // LANES):
            csl = pl.ds(c * LANES, LANES)
            idxb[sl][csl] = idxb[sl][csl] * 2

    for sl in range(3):
        wait_idx(sl, sl)
        double_idx(sl)
        fire_gather(sl, sl)

    def outer(k, carry):
        for sl in range(NB):
            i = k * NB + sl

            wait_gather(i, sl)

            @pl.when(i + NB < ipw)
            def _():
                fire_idx(i + NB, sl)

            @pl.when(i + 3 < ipw)
            def _():
                nsl = (sl + 3) % NB
                wait_idx(i + 3, nsl)
                for c in range(BT // LANES):
                    csl = pl.ds(c * LANES, LANES)
                    idxb[nsl][csl] = idxb[nsl][csl] * 2
                fire_gather(i + 3, nsl)

            @pl.when(i >= NB)
            def _():
                wait_out(i - NB, sl)

            g = base + i
            s = g // nbt
            rv = rows[sl]
            ov = outs[sl]
            pec = [pe_v[s, pl.ds(c * LANES, LANES)]
                   for c in range(MODEL_DIM // LANES)]

            @plsc.parallel_loop(0, BT, step=1, unroll=32)
            def row_body(r):
                rsplat = jnp.full((LANES,), r, jnp.int32)
                for c in range(MODEL_DIM // LANES):
                    v = rv[r, pl.ds(c * LANES, LANES)] + pec[c]
                    plsc.store_scatter(ov, [dvec[c], rsplat], v)

            fire_out(i, sl)
        return carry

    lax.fori_loop(0, ipw // NB, outer, 0)
    for sl in range(NB):
        wait_out(ipw - NB + sl, sl)


def kernel(x, table):
    b, seq = x.shape
    assert seq == MAX_LEN and b % BT == 0
    nbt = b // BT
    items = seq * nbt
    assert items % NW == 0
    ipw = items // NW
    assert ipw % NB == 0
    assert seq % 8 == 0
    xn4 = (x.T.astype(jnp.int32)
           .reshape(seq // 8, 8, nbt, BT).transpose(0, 2, 1, 3))
    pe = jnp.asarray(_pe_np())
    n_vocab = table.shape[0]
    tpad = jnp.pad(table, ((0, 0), (0, 128 - MODEL_DIM)))
    tpad2 = tpad.reshape(2 * n_vocab, MODEL_DIM)

    mesh = plsc.VectorSubcoreMesh(core_axis_name="c", subcore_axis_name="s")
    k = functools.partial(
        pl.kernel,
        mesh=mesh,
        out_type=jax.ShapeDtypeStruct((seq, MODEL_DIM // 8, nbt, 8, BT),
                                      jnp.float32),
        scratch_types=[
            pltpu.VMEM((8, BT), jnp.int32),
            pltpu.VMEM((MAX_LEN, MODEL_DIM), jnp.float32),
        ] + [pltpu.VMEM((BT,), jnp.int32)] * NB
          + [pltpu.VMEM((BT, MODEL_DIM), jnp.float32)] * NB
          + [pltpu.VMEM((MODEL_DIM, OVP), jnp.float32)] * NB
          + [pltpu.SemaphoreType.DMA] * (3 * NB),
        compiler_params=pltpu.CompilerParams(use_tc_tiling_on_sc=False,
                                             needs_layout_passes=False),
    )(functools.partial(_sc_body, ipw, nbt))
    out5 = k(xn4, pe, tpad2)
    return out5.transpose(2, 4, 0, 1, 3).reshape(b, seq, MODEL_DIM)


# speedup vs baseline: 1.1101x; 1.1101x over previous
import functools

import numpy as np
import jax
import jax.numpy as jnp
from jax import lax
from jax.experimental import pallas as pl
from jax.experimental.pallas import tpu as pltpu
from jax.experimental.pallas import tpu_sc as plsc

MAX_LEN = 200
MODEL_DIM = 64
NW = 32
NB = 4
LANES = 16
BT = 128
OVP = BT + 1


def _pe_np():
    pos = np.arange(MAX_LEN)[:, None]
    pe = pos / np.power(10000, 2.0 * np.arange(MODEL_DIM)[None, :] / MODEL_DIM)
    pe[:, 0::2] = np.sin(pe[:, 0::2])
    pe[:, 1::2] = np.cos(pe[:, 1::2])
    return pe.astype(np.float32)


def _sc_body(ipw, nbt, xn4_hbm, pe_hbm, table_hbm, out_hbm, idx_all, pe_v,
             i0, i1, i2, i3, r0, r1, r2, r3, o0, o1, o2, o3,
             is0, is1, is2, is3, gs0, gs1, gs2, gs3, os0, os1, os2, os3):
    del idx_all
    idxb = (i0, i1, i2, i3)
    rows = (r0, r1, r2, r3)
    outs = (o0, o1, o2, o3)
    isem = (is0, is1, is2, is3)
    gsem = (gs0, gs1, gs2, gs3)
    osem = (os0, os1, os2, os3)
    wid = lax.axis_index("s") * 2 + lax.axis_index("c")
    base = wid * ipw
    pltpu.sync_copy(pe_hbm, pe_v)
    iota = lax.iota(jnp.int32, LANES)
    dvec = [iota + (c * LANES) for c in range(MODEL_DIM // LANES)]

    def idx_src(i):
        g = base + i
        s, bt = g // nbt, g % nbt
        return xn4_hbm.at[s // 8, bt, s % 8]

    def fire_idx(i, sl):
        pltpu.async_copy(idx_src(i), idxb[sl], isem[sl])

    def wait_idx(i, sl):
        pltpu.make_async_copy(idx_src(i), idxb[sl], isem[sl]).wait()

    def fire_gather(i, sl):
        pltpu.async_copy(table_hbm.at[idxb[sl]], rows[sl], gsem[sl])

    def wait_gather(i, sl):
        pltpu.make_async_copy(table_hbm.at[idxb[sl]], rows[sl],
                              gsem[sl]).wait()

    def fire_out(i, sl):
        g = base + i
        s, bt = g // nbt, g % nbt
        for dt in range(MODEL_DIM // 8):
            pltpu.async_copy(outs[sl].at[pl.ds(dt * 8, 8), pl.ds(0, BT)],
                             out_hbm.at[s, dt, bt], osem[sl])

    def wait_out(i, sl):
        g = base + i
        s, bt = g // nbt, g % nbt
        for dt in range(MODEL_DIM // 8):
            pltpu.make_async_copy(outs[sl].at[pl.ds(dt * 8, 8), pl.ds(0, BT)],
                                  out_hbm.at[s, dt, bt], osem[sl]).wait()

    for sl in range(NB):
        fire_idx(sl, sl)
    def double_idx(sl):
        for c in range(BT // LANES):
            csl = pl.ds(c * LANES, LANES)
            idxb[sl][csl] = idxb[sl][csl] * 2

    for sl in range(3):
        wait_idx(sl, sl)
        double_idx(sl)
        fire_gather(sl, sl)

    def outer(k, carry):
        for sl in range(NB):
            i = k * NB + sl

            wait_gather(i, sl)

            @pl.when(i + NB < ipw)
            def _():
                fire_idx(i + NB, sl)

            @pl.when(i + 3 < ipw)
            def _():
                nsl = (sl + 3) % NB
                wait_idx(i + 3, nsl)
                for c in range(BT // LANES):
                    csl = pl.ds(c * LANES, LANES)
                    idxb[nsl][csl] = idxb[nsl][csl] * 2
                fire_gather(i + 3, nsl)

            @pl.when(i >= NB)
            def _():
                wait_out(i - NB, sl)

            g = base + i
            s = g // nbt
            rv = rows[sl]
            ov = outs[sl]
            pec = [pe_v[s, pl.ds(c * LANES, LANES)]
                   for c in range(MODEL_DIM // LANES)]

            @plsc.parallel_loop(0, BT, step=1, unroll=16)
            def row_body(r):
                rsplat = jnp.full((LANES,), r, jnp.int32)
                for c in range(MODEL_DIM // LANES):
                    v = rv[r, pl.ds(c * LANES, LANES)] + pec[c]
                    plsc.store_scatter(ov, [dvec[c], rsplat], v)

            fire_out(i, sl)
        return carry

    lax.fori_loop(0, ipw // NB, outer, 0)
    for sl in range(NB):
        wait_out(ipw - NB + sl, sl)


def kernel(x, table):
    b, seq = x.shape
    assert seq == MAX_LEN and b % BT == 0
    nbt = b // BT
    items = seq * nbt
    assert items % NW == 0
    ipw = items // NW
    assert ipw % NB == 0
    assert seq % 8 == 0
    xn4 = (x.T.astype(jnp.int32)
           .reshape(seq // 8, 8, nbt, BT).transpose(0, 2, 1, 3))
    pe = jnp.asarray(_pe_np())
    n_vocab = table.shape[0]
    tpad = jnp.pad(table, ((0, 0), (0, 128 - MODEL_DIM)))
    tpad2 = tpad.reshape(2 * n_vocab, MODEL_DIM)

    mesh = plsc.VectorSubcoreMesh(core_axis_name="c", subcore_axis_name="s")
    k = functools.partial(
        pl.kernel,
        mesh=mesh,
        out_type=jax.ShapeDtypeStruct((seq, MODEL_DIM // 8, nbt, 8, BT),
                                      jnp.float32),
        scratch_types=[
            pltpu.VMEM((8, BT), jnp.int32),
            pltpu.VMEM((MAX_LEN, MODEL_DIM), jnp.float32),
        ] + [pltpu.VMEM((BT,), jnp.int32)] * NB
          + [pltpu.VMEM((BT, MODEL_DIM), jnp.float32)] * NB
          + [pltpu.VMEM((MODEL_DIM, OVP), jnp.float32)] * NB
          + [pltpu.SemaphoreType.DMA] * (3 * NB),
        compiler_params=pltpu.CompilerParams(use_tc_tiling_on_sc=False,
                                             needs_layout_passes=False),
    )(functools.partial(_sc_body, ipw, nbt))
    out5 = k(xn4, pe, tpad2)
    return out5.transpose(2, 4, 0, 1, 3).reshape(b, seq, MODEL_DIM)
